# baseline (speedup 1.0000x reference)
import functools

import jax
import jax.numpy as jnp
from jax import lax
from jax.experimental import pallas as pl
from jax.experimental.pallas import tpu as pltpu
from jax.experimental.pallas import tpu_sc as plsc

N = 10000
E = 320000
F_IN = 128
H = 64
Z = 32

NC = 2
NS = 16
NW = NC * NS
CHUNK = 80
EPT = E // NW
ITERS = EPT // CHUNK
WTILES = 16
RPT = N // WTILES

_R = 2000
_GRID = N // _R

def _mesh():
    return plsc.VectorSubcoreMesh(
        core_axis_name="c", subcore_axis_name="s", num_cores=NC, num_subcores=NS)


NBUF = 5


def _seg_body(W, g_hbm, src_hbm, dst_hbm, zeros_hbm, out_hbm,
              idxs, idxd, r0, r1, r2, r3, r4, acc,
              sg0, sg1, sg2, sg3, sg4, ss0, ss1, ss2, ss3, ss4):
    rows = [r0, r1, r2, r3, r4]
    sem_g = [sg0, sg1, sg2, sg3, sg4]
    sem_s = [ss0, ss1, ss2, ss3, ss4]
    c = lax.axis_index("c")
    s = lax.axis_index("s")

    pltpu.sync_copy(zeros_hbm, acc.at[pl.ds(s * RPT, RPT)])

    rbase = (c * NS + s) * ITERS
    pltpu.sync_copy(src_hbm.at[pl.ds(rbase, ITERS)], idxs)
    pltpu.sync_copy(dst_hbm.at[pl.ds(rbase, ITERS)], idxd)
    plsc.subcore_barrier()

    for t in range(NBUF):
        pltpu.async_copy(g_hbm.at[idxs.at[t]], rows[t], sem_g[t])

    def body(j, carry):
        cb = NBUF * j
        for t in range(NBUF):
            ch = cb + t
            pltpu.make_async_copy(g_hbm.at[idxs.at[ch]], rows[t], sem_g[t]).wait()
            pltpu.async_copy(rows[t], acc.at[idxd.at[ch]], sem_s[t], add=True)
        for t in range(NBUF):
            ch = cb + t
            pltpu.make_async_copy(rows[t], acc.at[idxd.at[ch]], sem_s[t]).wait()
            pltpu.async_copy(g_hbm.at[idxs.at[ch + NBUF]], rows[t], sem_g[t])
        return carry

    lax.fori_loop(0, ITERS // NBUF - 1, body, 0)
    cb = ITERS - NBUF
    for t in range(NBUF):
        pltpu.make_async_copy(g_hbm.at[idxs.at[cb + t]], rows[t], sem_g[t]).wait()
        pltpu.async_copy(rows[t], acc.at[idxd.at[cb + t]], sem_s[t], add=True)
    for t in range(NBUF):
        pltpu.make_async_copy(rows[t], acc.at[idxd.at[cb + t]], sem_s[t]).wait()
    plsc.subcore_barrier()

    pltpu.sync_copy(acc.at[pl.ds(s * RPT, RPT)],
                    out_hbm.at[c, pl.ds(s * RPT, RPT)])


_SC_PARAMS = pltpu.CompilerParams(use_tc_tiling_on_sc=False)


def _make_seg(W):
    return pl.kernel(
        functools.partial(_seg_body, W),
        out_type=jax.ShapeDtypeStruct((NC, N, W), jnp.bfloat16),
        mesh=_mesh(),
        compiler_params=_SC_PARAMS,
        scratch_types=[
            pltpu.VMEM((ITERS, CHUNK), jnp.int32),
            pltpu.VMEM((ITERS, CHUNK), jnp.int32),
            *[pltpu.VMEM((CHUNK, W), jnp.bfloat16) for _ in range(NBUF)],
            pltpu.VMEM_SHARED((N, W), jnp.bfloat16),
            *[pltpu.SemaphoreType.DMA for _ in range(2 * NBUF)],
        ],
    )


def _deg_body(dst_hbm, zeros_hbm, ones_hbm, out_hbm, idxd, rows, acc, sem):
    c = lax.axis_index("c")
    s = lax.axis_index("s")

    pltpu.sync_copy(zeros_hbm, acc.at[pl.ds(s * RPT, RPT)])

    rbase = (c * NS + s) * ITERS
    pltpu.sync_copy(dst_hbm.at[pl.ds(rbase, ITERS)], idxd)
    pltpu.sync_copy(ones_hbm, rows)
    plsc.subcore_barrier()

    pltpu.async_copy(rows, acc.at[idxd.at[0]], sem, add=True)

    def body(j, carry):
        pltpu.async_copy(rows, acc.at[idxd.at[j + 1]], sem, add=True)
        pltpu.make_async_copy(rows, acc.at[idxd.at[j]], sem).wait()
        return carry

    lax.fori_loop(0, ITERS - 1, body, 0)
    pltpu.make_async_copy(rows, acc.at[idxd.at[ITERS - 1]], sem).wait()
    plsc.subcore_barrier()

    pltpu.sync_copy(acc.at[pl.ds(s * RPT, RPT)],
                    out_hbm.at[c, pl.ds(s * RPT, RPT)])


def _make_deg():
    return pl.kernel(
        _deg_body,
        out_type=jax.ShapeDtypeStruct((NC, N, 16), jnp.bfloat16),
        mesh=_mesh(),
        compiler_params=_SC_PARAMS,
        scratch_types=[
            pltpu.VMEM((ITERS, CHUNK), jnp.int32),
            pltpu.VMEM((CHUNK, 16), jnp.bfloat16),
            pltpu.VMEM_SHARED((N, 16), jnp.bfloat16),
            pltpu.SemaphoreType.DMA,
        ],
    )


def _mm(a, b):
    return jax.lax.dot_general(a.astype(jnp.bfloat16), b.astype(jnp.bfloat16),
                               (((1,), (0,)), ((), ())),
                               preferred_element_type=jnp.float32)


def _f32(aggp):
    return aggp[0].astype(jnp.float32) + aggp[1].astype(jnp.float32)


def _k1_body(feat, W1, b1, degp, x_o, g0_o, dinv_o):
    d = degp[0].astype(jnp.float32) + degp[1].astype(jnp.float32)
    dinv = lax.rsqrt(jnp.maximum(d[:, :1], 1.0))
    x = jnp.maximum(_mm(feat[...], W1[...]) + b1[...], 0.0)
    x_o[...] = x.astype(jnp.bfloat16)
    g0_o[...] = (x * dinv).astype(jnp.bfloat16)
    dinv_o[...] = jnp.broadcast_to(dinv, dinv_o.shape)


def _k2_body(x, aggp, dinv, f1_o, g1_o):
    dv = dinv[:, :1]
    f1 = x[...].astype(jnp.float32) - _f32(aggp) * dv
    f1_o[...] = f1.astype(jnp.bfloat16)
    g1_o[...] = (f1 * dv).astype(jnp.bfloat16)


def _k3_body(x, f1, aggp, dinv, eps, A0, A1, A2, b2, repW, repb, recW, recb,
             dW1, db1, xd_o, g0d_o, kl_o):
    dv = dinv[:, :1]
    f2 = f1[...].astype(jnp.float32) - _f32(aggp) * dv
    pos = _mm(x[...], A0[...]) + _mm(f1[...], A1[...]) + _mm(f2, A2[...]) + b2[...]
    mu = _mm(pos, repW[...]) + repb[...]
    expmu = jnp.exp(mu)
    z = mu + eps[...].astype(jnp.float32) * jnp.exp(mu * 0.5)
    y = _mm(z, recW[...]) + recb[...]
    xd = jnp.maximum(_mm(y, dW1[...]) + db1[...], 0.0)
    xd_o[...] = xd.astype(jnp.bfloat16)
    g0d_o[...] = (xd * dv).astype(jnp.bfloat16)

    @pl.when(pl.program_id(0) == 0)
    def _():
        kl_o[...] = jnp.zeros((1, 1), jnp.float32)

    kl_o[...] += jnp.sum(1.0 + mu - mu * mu - expmu).reshape(1, 1)


def _k4_body(xd, aggp, dinv, f1d_o, g1d_o):
    dv = dinv[:, :1]
    f1d = xd[...].astype(jnp.float32) - _f32(aggp) * dv
    f1d_o[...] = f1d.astype(jnp.bfloat16)
    g1d_o[...] = (f1d * dv).astype(jnp.bfloat16)


def _k5_body(xd, f1d, aggp, dinv, feat, B0, B1, B2, b2, rec_o):
    dv = dinv[:, :1]
    f2d = f1d[...].astype(jnp.float32) - _f32(aggp) * dv
    xr = _mm(xd[...], B0[...]) + _mm(f1d[...], B1[...]) + _mm(f2d, B2[...]) + b2[...]
    r = xr - feat[...]

    @pl.when(pl.program_id(0) == 0)
    def _():
        rec_o[...] = jnp.zeros((1, 1), jnp.float32)

    rec_o[...] += jnp.sum(r * r).reshape(1, 1)


def _row_spec(w):
    return pl.BlockSpec((_R, w), lambda i: (i, 0))


def _row3_spec(w):
    return pl.BlockSpec((NC, _R, w), lambda i: (0, i, 0))


def _full_spec(shape):
    nd = len(shape)
    if nd == 1:
        return pl.BlockSpec(shape, lambda i: (0,))
    return pl.BlockSpec(shape, lambda i: (0,) * nd)


def _scalar_spec():
    return pl.BlockSpec((1, 1), lambda i: (0, 0))


def _sds(shape, dtype=jnp.float32):
    return jax.ShapeDtypeStruct(shape, dtype)


def _combine(W2, h):
    Wa, Wb, Wc = W2[:h], W2[h:2 * h], W2[2 * h:]
    return 3.0 * Wa, -3.0 * Wa + 3.0 * Wb, 0.75 * Wa - 1.5 * Wb + 0.75 * Wc


def kernel(features, edge_index, enc_W1, enc_b1, enc_W2, enc_b2, rep_W, rep_b,
           rec_W, rec_b, dec_W1, dec_b1, dec_W2, dec_b2, disc_W):
    src = edge_index[0].reshape(E // CHUNK, CHUNK)
    dst = edge_index[1].reshape(E // CHUNK, CHUNK)
    zeros16 = jnp.zeros((RPT, 16), jnp.bfloat16)
    zeros64 = jnp.zeros((RPT, H), jnp.bfloat16)
    zeros128 = jnp.zeros((RPT, F_IN), jnp.bfloat16)
    ones16 = jnp.ones((CHUNK, 16), jnp.bfloat16)

    A0, A1, A2 = _combine(enc_W2, H)
    B0, B1, B2 = _combine(dec_W2, F_IN)
    keps = jax.random.split(jax.random.key(42))[1]
    eps = jax.random.normal(keps, (N, Z), dtype=jnp.float32).astype(jnp.bfloat16)

    seg64 = _make_seg(H)
    seg128 = _make_seg(F_IN)

    degp = _make_deg()(dst, zeros16, ones16)

    x, g0, dinv = pl.pallas_call(
        _k1_body,
        grid=(_GRID,),
        in_specs=[_row_spec(F_IN), _full_spec((F_IN, H)), _full_spec((1, H)),
                  _row3_spec(16)],
        out_specs=[_row_spec(H), _row_spec(H), _row_spec(F_IN)],
        out_shape=[_sds((N, H), jnp.bfloat16), _sds((N, H), jnp.bfloat16),
                   _sds((N, F_IN))],
    )(features, enc_W1, enc_b1.reshape(1, H), degp)

    aggp = seg64(g0, src, dst, zeros64)

    f1, g1 = pl.pallas_call(
        _k2_body,
        grid=(_GRID,),
        in_specs=[_row_spec(H), _row3_spec(H), _row_spec(F_IN)],
        out_specs=[_row_spec(H), _row_spec(H)],
        out_shape=[_sds((N, H), jnp.bfloat16), _sds((N, H), jnp.bfloat16)],
    )(x, aggp, dinv)

    aggp2 = seg64(g1, src, dst, zeros64)

    xd, g0d, kls = pl.pallas_call(
        _k3_body,
        grid=(_GRID,),
        in_specs=[_row_spec(H), _row_spec(H), _row3_spec(H), _row_spec(F_IN),
                  _row_spec(Z),
                  _full_spec((H, H)), _full_spec((H, H)), _full_spec((H, H)),
                  _full_spec((1, H)), _full_spec((H, Z)), _full_spec((1, Z)),
                  _full_spec((Z, H)), _full_spec((1, H)), _full_spec((H, F_IN)),
                  _full_spec((1, F_IN))],
        out_specs=[_row_spec(F_IN), _row_spec(F_IN), _scalar_spec()],
        out_shape=[_sds((N, F_IN), jnp.bfloat16), _sds((N, F_IN), jnp.bfloat16),
                   _sds((1, 1))],
    )(x, f1, aggp2, dinv, eps, A0, A1, A2, enc_b2.reshape(1, H),
      rep_W, rep_b.reshape(1, Z), rec_W, rec_b.reshape(1, H),
      dec_W1, dec_b1.reshape(1, F_IN))

    aggp3 = seg128(g0d, src, dst, zeros128)

    f1d, g1d = pl.pallas_call(
        _k4_body,
        grid=(_GRID,),
        in_specs=[_row_spec(F_IN), _row3_spec(F_IN), _row_spec(F_IN)],
        out_specs=[_row_spec(F_IN), _row_spec(F_IN)],
        out_shape=[_sds((N, F_IN), jnp.bfloat16), _sds((N, F_IN), jnp.bfloat16)],
    )(xd, aggp3, dinv)

    aggp4 = seg128(g1d, src, dst, zeros128)

    recs = pl.pallas_call(
        _k5_body,
        grid=(_GRID,),
        in_specs=[_row_spec(F_IN), _row_spec(F_IN), _row3_spec(F_IN),
                  _row_spec(F_IN), _row_spec(F_IN),
                  _full_spec((F_IN, F_IN)), _full_spec((F_IN, F_IN)),
                  _full_spec((F_IN, F_IN)), _full_spec((1, F_IN))],
        out_specs=_scalar_spec(),
        out_shape=_sds((1, 1)),
    )(xd, f1d, aggp4, dinv, features, B0, B1, B2, dec_b2.reshape(1, F_IN))

    return recs[0, 0] - 0.5 * kls[0, 0]


# speedup vs baseline: 1.0551x; 1.0551x over previous
import functools

import jax
import jax.numpy as jnp
from jax import lax
from jax.experimental import pallas as pl
from jax.experimental.pallas import tpu as pltpu
from jax.experimental.pallas import tpu_sc as plsc

N = 10000
E = 320000
F_IN = 128
H = 64
Z = 32

NC = 2
NS = 16
NW = NC * NS
CHUNK = 80
EPT = E // NW
ITERS = EPT // CHUNK
WTILES = 16
RPT = N // WTILES

_R = 2000
_GRID = N // _R

def _mesh():
    return plsc.VectorSubcoreMesh(
        core_axis_name="c", subcore_axis_name="s", num_cores=NC, num_subcores=NS)


NBUF = 5


def _seg_body(W, g_hbm, src_hbm, dst_hbm, zeros_hbm, out_hbm,
              idxs, idxd, r0, r1, r2, r3, r4, acc,
              sg0, sg1, sg2, sg3, sg4, ss0, ss1, ss2, ss3, ss4):
    rows = [r0, r1, r2, r3, r4]
    sem_g = [sg0, sg1, sg2, sg3, sg4]
    sem_s = [ss0, ss1, ss2, ss3, ss4]
    c = lax.axis_index("c")
    s = lax.axis_index("s")

    pltpu.sync_copy(zeros_hbm, acc.at[pl.ds(s * RPT, RPT)])

    rbase = (c * NS + s) * ITERS
    pltpu.sync_copy(src_hbm.at[pl.ds(rbase, ITERS)], idxs)
    pltpu.sync_copy(dst_hbm.at[pl.ds(rbase, ITERS)], idxd)
    plsc.subcore_barrier()

    for t in range(NBUF):
        pltpu.async_copy(g_hbm.at[idxs.at[t]], rows[t], sem_g[t])

    def body(j, carry):
        cb = NBUF * j
        for t in range(NBUF):
            ch = cb + t
            pltpu.make_async_copy(g_hbm.at[idxs.at[ch]], rows[t], sem_g[t]).wait()
            pltpu.async_copy(rows[t], acc.at[idxd.at[ch]], sem_s[t], add=True)
            pltpu.make_async_copy(rows[t], acc.at[idxd.at[ch]], sem_s[t]).wait()
            pltpu.async_copy(g_hbm.at[idxs.at[ch + NBUF]], rows[t], sem_g[t])
        return carry

    lax.fori_loop(0, ITERS // NBUF - 1, body, 0)
    cb = ITERS - NBUF
    for t in range(NBUF):
        pltpu.make_async_copy(g_hbm.at[idxs.at[cb + t]], rows[t], sem_g[t]).wait()
        pltpu.async_copy(rows[t], acc.at[idxd.at[cb + t]], sem_s[t], add=True)
    for t in range(NBUF):
        pltpu.make_async_copy(rows[t], acc.at[idxd.at[cb + t]], sem_s[t]).wait()
    plsc.subcore_barrier()

    pltpu.sync_copy(acc.at[pl.ds(s * RPT, RPT)],
                    out_hbm.at[c, pl.ds(s * RPT, RPT)])


_SC_PARAMS = pltpu.CompilerParams(use_tc_tiling_on_sc=False)


def _make_seg(W):
    return pl.kernel(
        functools.partial(_seg_body, W),
        out_type=jax.ShapeDtypeStruct((NC, N, W), jnp.bfloat16),
        mesh=_mesh(),
        compiler_params=_SC_PARAMS,
        scratch_types=[
            pltpu.VMEM((ITERS, CHUNK), jnp.int32),
            pltpu.VMEM((ITERS, CHUNK), jnp.int32),
            *[pltpu.VMEM((CHUNK, W), jnp.bfloat16) for _ in range(NBUF)],
            pltpu.VMEM_SHARED((N, W), jnp.bfloat16),
            *[pltpu.SemaphoreType.DMA for _ in range(2 * NBUF)],
        ],
    )


def _deg_body(dst_hbm, zeros_hbm, ones_hbm, out_hbm, idxd, rows, acc, sem):
    c = lax.axis_index("c")
    s = lax.axis_index("s")

    pltpu.sync_copy(zeros_hbm, acc.at[pl.ds(s * RPT, RPT)])

    rbase = (c * NS + s) * ITERS
    pltpu.sync_copy(dst_hbm.at[pl.ds(rbase, ITERS)], idxd)
    pltpu.sync_copy(ones_hbm, rows)
    plsc.subcore_barrier()

    pltpu.async_copy(rows, acc.at[idxd.at[0]], sem, add=True)

    def body(j, carry):
        pltpu.async_copy(rows, acc.at[idxd.at[j + 1]], sem, add=True)
        pltpu.make_async_copy(rows, acc.at[idxd.at[j]], sem).wait()
        return carry

    lax.fori_loop(0, ITERS - 1, body, 0)
    pltpu.make_async_copy(rows, acc.at[idxd.at[ITERS - 1]], sem).wait()
    plsc.subcore_barrier()

    pltpu.sync_copy(acc.at[pl.ds(s * RPT, RPT)],
                    out_hbm.at[c, pl.ds(s * RPT, RPT)])


def _make_deg():
    return pl.kernel(
        _deg_body,
        out_type=jax.ShapeDtypeStruct((NC, N, 16), jnp.bfloat16),
        mesh=_mesh(),
        compiler_params=_SC_PARAMS,
        scratch_types=[
            pltpu.VMEM((ITERS, CHUNK), jnp.int32),
            pltpu.VMEM((CHUNK, 16), jnp.bfloat16),
            pltpu.VMEM_SHARED((N, 16), jnp.bfloat16),
            pltpu.SemaphoreType.DMA,
        ],
    )


def _mm(a, b):
    return jax.lax.dot_general(a.astype(jnp.bfloat16), b.astype(jnp.bfloat16),
                               (((1,), (0,)), ((), ())),
                               preferred_element_type=jnp.float32)


def _f32(aggp):
    return aggp[0].astype(jnp.float32) + aggp[1].astype(jnp.float32)


def _k1_body(feat, W1, b1, degp, x_o, g0_o, dinv_o):
    d = degp[0].astype(jnp.float32) + degp[1].astype(jnp.float32)
    dinv = lax.rsqrt(jnp.maximum(d[:, :1], 1.0))
    x = jnp.maximum(_mm(feat[...], W1[...]) + b1[...], 0.0)
    x_o[...] = x.astype(jnp.bfloat16)
    g0_o[...] = (x * dinv).astype(jnp.bfloat16)
    dinv_o[...] = jnp.broadcast_to(dinv, dinv_o.shape)


def _k2_body(x, aggp, dinv, f1_o, g1_o):
    dv = dinv[:, :1]
    f1 = x[...].astype(jnp.float32) - _f32(aggp) * dv
    f1_o[...] = f1.astype(jnp.bfloat16)
    g1_o[...] = (f1 * dv).astype(jnp.bfloat16)


def _k3_body(x, f1, aggp, dinv, eps, A0, A1, A2, b2, repW, repb, recW, recb,
             dW1, db1, xd_o, g0d_o, kl_o):
    dv = dinv[:, :1]
    f2 = f1[...].astype(jnp.float32) - _f32(aggp) * dv
    pos = _mm(x[...], A0[...]) + _mm(f1[...], A1[...]) + _mm(f2, A2[...]) + b2[...]
    mu = _mm(pos, repW[...]) + repb[...]
    expmu = jnp.exp(mu)
    z = mu + eps[...].astype(jnp.float32) * jnp.exp(mu * 0.5)
    y = _mm(z, recW[...]) + recb[...]
    xd = jnp.maximum(_mm(y, dW1[...]) + db1[...], 0.0)
    xd_o[...] = xd.astype(jnp.bfloat16)
    g0d_o[...] = (xd * dv).astype(jnp.bfloat16)

    @pl.when(pl.program_id(0) == 0)
    def _():
        kl_o[...] = jnp.zeros((1, 1), jnp.float32)

    kl_o[...] += jnp.sum(1.0 + mu - mu * mu - expmu).reshape(1, 1)


def _k4_body(xd, aggp, dinv, f1d_o, g1d_o):
    dv = dinv[:, :1]
    f1d = xd[...].astype(jnp.float32) - _f32(aggp) * dv
    f1d_o[...] = f1d.astype(jnp.bfloat16)
    g1d_o[...] = (f1d * dv).astype(jnp.bfloat16)


def _k5_body(xd, f1d, aggp, dinv, feat, B0, B1, B2, b2, rec_o):
    dv = dinv[:, :1]
    f2d = f1d[...].astype(jnp.float32) - _f32(aggp) * dv
    xr = _mm(xd[...], B0[...]) + _mm(f1d[...], B1[...]) + _mm(f2d, B2[...]) + b2[...]
    r = xr - feat[...]

    @pl.when(pl.program_id(0) == 0)
    def _():
        rec_o[...] = jnp.zeros((1, 1), jnp.float32)

    rec_o[...] += jnp.sum(r * r).reshape(1, 1)


def _row_spec(w):
    return pl.BlockSpec((_R, w), lambda i: (i, 0))


def _row3_spec(w):
    return pl.BlockSpec((NC, _R, w), lambda i: (0, i, 0))


def _full_spec(shape):
    nd = len(shape)
    if nd == 1:
        return pl.BlockSpec(shape, lambda i: (0,))
    return pl.BlockSpec(shape, lambda i: (0,) * nd)


def _scalar_spec():
    return pl.BlockSpec((1, 1), lambda i: (0, 0))


def _sds(shape, dtype=jnp.float32):
    return jax.ShapeDtypeStruct(shape, dtype)


def _combine(W2, h):
    Wa, Wb, Wc = W2[:h], W2[h:2 * h], W2[2 * h:]
    return 3.0 * Wa, -3.0 * Wa + 3.0 * Wb, 0.75 * Wa - 1.5 * Wb + 0.75 * Wc


def kernel(features, edge_index, enc_W1, enc_b1, enc_W2, enc_b2, rep_W, rep_b,
           rec_W, rec_b, dec_W1, dec_b1, dec_W2, dec_b2, disc_W):
    src = edge_index[0].reshape(E // CHUNK, CHUNK)
    dst = edge_index[1].reshape(E // CHUNK, CHUNK)
    zeros16 = jnp.zeros((RPT, 16), jnp.bfloat16)
    zeros64 = jnp.zeros((RPT, H), jnp.bfloat16)
    zeros128 = jnp.zeros((RPT, F_IN), jnp.bfloat16)
    ones16 = jnp.ones((CHUNK, 16), jnp.bfloat16)

    A0, A1, A2 = _combine(enc_W2, H)
    B0, B1, B2 = _combine(dec_W2, F_IN)
    keps = jax.random.split(jax.random.key(42))[1]
    eps = jax.random.normal(keps, (N, Z), dtype=jnp.float32).astype(jnp.bfloat16)

    seg64 = _make_seg(H)
    seg128 = _make_seg(F_IN)

    degp = _make_deg()(dst, zeros16, ones16)

    x, g0, dinv = pl.pallas_call(
        _k1_body,
        grid=(_GRID,),
        in_specs=[_row_spec(F_IN), _full_spec((F_IN, H)), _full_spec((1, H)),
                  _row3_spec(16)],
        out_specs=[_row_spec(H), _row_spec(H), _row_spec(F_IN)],
        out_shape=[_sds((N, H), jnp.bfloat16), _sds((N, H), jnp.bfloat16),
                   _sds((N, F_IN))],
    )(features, enc_W1, enc_b1.reshape(1, H), degp)

    aggp = seg64(g0, src, dst, zeros64)

    f1, g1 = pl.pallas_call(
        _k2_body,
        grid=(_GRID,),
        in_specs=[_row_spec(H), _row3_spec(H), _row_spec(F_IN)],
        out_specs=[_row_spec(H), _row_spec(H)],
        out_shape=[_sds((N, H), jnp.bfloat16), _sds((N, H), jnp.bfloat16)],
    )(x, aggp, dinv)

    aggp2 = seg64(g1, src, dst, zeros64)

    xd, g0d, kls = pl.pallas_call(
        _k3_body,
        grid=(_GRID,),
        in_specs=[_row_spec(H), _row_spec(H), _row3_spec(H), _row_spec(F_IN),
                  _row_spec(Z),
                  _full_spec((H, H)), _full_spec((H, H)), _full_spec((H, H)),
                  _full_spec((1, H)), _full_spec((H, Z)), _full_spec((1, Z)),
                  _full_spec((Z, H)), _full_spec((1, H)), _full_spec((H, F_IN)),
                  _full_spec((1, F_IN))],
        out_specs=[_row_spec(F_IN), _row_spec(F_IN), _scalar_spec()],
        out_shape=[_sds((N, F_IN), jnp.bfloat16), _sds((N, F_IN), jnp.bfloat16),
                   _sds((1, 1))],
    )(x, f1, aggp2, dinv, eps, A0, A1, A2, enc_b2.reshape(1, H),
      rep_W, rep_b.reshape(1, Z), rec_W, rec_b.reshape(1, H),
      dec_W1, dec_b1.reshape(1, F_IN))

    aggp3 = seg128(g0d, src, dst, zeros128)

    f1d, g1d = pl.pallas_call(
        _k4_body,
        grid=(_GRID,),
        in_specs=[_row_spec(F_IN), _row3_spec(F_IN), _row_spec(F_IN)],
        out_specs=[_row_spec(F_IN), _row_spec(F_IN)],
        out_shape=[_sds((N, F_IN), jnp.bfloat16), _sds((N, F_IN), jnp.bfloat16)],
    )(xd, aggp3, dinv)

    aggp4 = seg128(g1d, src, dst, zeros128)

    recs = pl.pallas_call(
        _k5_body,
        grid=(_GRID,),
        in_specs=[_row_spec(F_IN), _row_spec(F_IN), _row3_spec(F_IN),
                  _row_spec(F_IN), _row_spec(F_IN),
                  _full_spec((F_IN, F_IN)), _full_spec((F_IN, F_IN)),
                  _full_spec((F_IN, F_IN)), _full_spec((1, F_IN))],
        out_specs=_scalar_spec(),
        out_shape=_sds((1, 1)),
    )(xd, f1d, aggp4, dinv, features, B0, B1, B2, dec_b2.reshape(1, F_IN))

    return recs[0, 0] - 0.5 * kls[0, 0]
